# baseline (speedup 1.0000x reference)
import functools

import jax
import jax.numpy as jnp
from jax import lax
from jax.experimental import pallas as pl
from jax.experimental.pallas import tpu as pltpu
from jax.experimental.pallas import tpu_sc as plsc

N = 10000
E = 160000
STATE = 32
T = 3

NC = 2
NS = 16
NW = NC * NS
PER_W = E // NW
CHUNK = 704
NFULL = PER_W // CHUNK
TAIL = PER_W - NFULL * CHUNK
ROWS_PER_TILE = N // NS

_mesh = plsc.VectorSubcoreMesh(core_axis_name="c", subcore_axis_name="s")
_sc_params = pltpu.CompilerParams(use_tc_tiling_on_sc=False)

_TP = (9.99976908e-01, 1.16046247e-01, 1.60365303e-03)
_TQ = (4.49318701e-01, 1.80974753e-02, 6.24348775e-05)


def _tanh16(x):
    xc = jnp.minimum(jnp.maximum(x, -4.8), 4.8)
    u = xc * xc
    p = _TP[0] + u * (_TP[1] + u * _TP[2])
    q = 1.0 + u * (_TQ[0] + u * (_TQ[1] + u * _TQ[2]))
    return xc * p / q


def _edge_compute(qr, bb, ewv, nedges):
    def do_edge(e, w_scalar):
        w = jnp.full((16,), w_scalar, jnp.float32)
        for h in range(STATE // 16):
            x = qr[e, pl.ds(16 * h, 16)] + bb[e, pl.ds(16 * h, 16)]
            bb[e, pl.ds(16 * h, 16)] = _tanh16(x) * w

    def body(g, _):
        wv = ewv[pl.ds(16 * g, 16)]
        for i in range(16):
            do_edge(16 * g + i, wv[i])
        return 0

    lax.fori_loop(0, nedges // 16, body, 0, unroll=False)
    if nedges % 16:
        tb = nedges - 16
        wv = ewv[pl.ds(tb, 16)]
        for i in range(16 - nedges % 16, 16):
            do_edge(tb + i, wv[i])


@functools.partial(
    pl.kernel,
    out_type=jax.ShapeDtypeStruct((NC, N, STATE), jnp.float32),
    mesh=_mesh,
    compiler_params=_sc_params,
    scratch_types=[
        [pltpu.VMEM((CHUNK,), jnp.int32)] * 2,
        [pltpu.VMEM((CHUNK,), jnp.int32)] * 2,
        [pltpu.VMEM((CHUNK,), jnp.float32)] * 2,
        [pltpu.VMEM((CHUNK, STATE), jnp.float32)] * 2,
        [pltpu.VMEM((CHUNK, STATE), jnp.float32)] * 2,
        pltpu.VMEM((TAIL,), jnp.int32),
        pltpu.VMEM((TAIL,), jnp.int32),
        pltpu.VMEM((TAIL,), jnp.float32),
        pltpu.VMEM((TAIL, STATE), jnp.float32),
        pltpu.VMEM((TAIL, STATE), jnp.float32),
        pltpu.VMEM_SHARED((N, STATE), jnp.float32),
        [pltpu.SemaphoreType.DMA] * 2,
        [pltpu.SemaphoreType.DMA] * 2,
    ],
)
def _sc_edge_pass(q_hbm, b_hbm, src_hbm, dst_hbm, ew_hbm, zeros_hbm, out_hbm,
                  sidx_v, didx_v, ew_v, qrows_v, b_v,
                  sidx_t, didx_t, ew_t, qrows_t, b_t,
                  acc_sh, sem_in, sem_sc):
    cid = lax.axis_index("c")
    sid = lax.axis_index("s")
    wid = sid * NC + cid
    base = wid * PER_W
    rbase = sid * ROWS_PER_TILE

    def stage(ci, p):
        off = base + ci * CHUNK
        pltpu.sync_copy(src_hbm.at[pl.ds(off, CHUNK)], sidx_v[p])
        gat = pltpu.async_copy(q_hbm.at[sidx_v[p]], qrows_v[p], sem_in[p])
        bcp = pltpu.async_copy(b_hbm.at[pl.ds(off, CHUNK)], b_v[p], sem_in[p])
        dcp = pltpu.async_copy(dst_hbm.at[pl.ds(off, CHUNK)], didx_v[p], sem_in[p])
        ecp = pltpu.async_copy(ew_hbm.at[pl.ds(off, CHUNK)], ew_v[p], sem_in[p])
        return (gat, bcp, dcp, ecp)

    pend_in = [None, None]
    pend_sc = [None, None]
    pend_in[0] = stage(0, 0)
    pltpu.sync_copy(zeros_hbm.at[pl.ds(rbase, ROWS_PER_TILE)],
                    acc_sh.at[pl.ds(rbase, ROWS_PER_TILE)])
    plsc.subcore_barrier()
    for ci in range(NFULL):
        p = ci % 2
        q = 1 - p
        if ci + 1 < NFULL:
            if pend_sc[q] is not None:
                pend_sc[q].wait()
                pend_sc[q] = None
            pend_in[q] = stage(ci + 1, q)
        for d in pend_in[p]:
            d.wait()
        pend_in[p] = None
        _edge_compute(qrows_v[p], b_v[p], ew_v[p], CHUNK)
        pend_sc[p] = pltpu.async_copy(b_v[p], acc_sh.at[didx_v[p]],
                                      sem_sc[p], add=True)
    toff = base + NFULL * CHUNK
    pltpu.sync_copy(src_hbm.at[pl.ds(toff, TAIL)], sidx_t)
    pltpu.sync_copy(dst_hbm.at[pl.ds(toff, TAIL)], didx_t)
    pltpu.sync_copy(ew_hbm.at[pl.ds(toff, TAIL)], ew_t)
    pltpu.sync_copy(b_hbm.at[pl.ds(toff, TAIL)], b_t)
    pltpu.async_copy(q_hbm.at[sidx_t], qrows_t, sem_in[0]).wait()
    _edge_compute(qrows_t, b_t, ew_t, TAIL)
    pltpu.sync_copy(b_t, acc_sh.at[didx_t], add=True)
    for p in range(2):
        if pend_sc[p] is not None:
            pend_sc[p].wait()
    plsc.subcore_barrier()
    pltpu.sync_copy(acc_sh.at[pl.ds(rbase, ROWS_PER_TILE)],
                    out_hbm.at[cid, pl.ds(rbase, ROWS_PER_TILE)])


def _gnn_layer(x, arcs, src, dst, ew, Wm, Ws, bs, Wo, mask, zeros_acc):
    d = x.shape[1]
    Wm_s, Wm_x, Wm_a = Wm[:STATE], Wm[STATE:STATE + d], Wm[STATE + d:]
    Ws_x, Ws_a = Ws[:d], Ws[d:]
    A = x @ Wm_x
    B = arcs @ Wm_a
    nb = x @ Ws_x + bs
    state = jnp.zeros((N, STATE), jnp.float32)
    Q = A
    for _ in range(T):
        parts = _sc_edge_pass(Q, B, src, dst, ew, zeros_acc)
        agg = parts[0] + parts[1]
        state = jnp.tanh(nb + agg @ Ws_a)
        Q = A + state @ Wm_s
    out = jnp.where(mask[:, None], state @ Wo, 0.0)
    return state, out


def kernel(nodes, arcs, edge_index, edge_weights, set_mask, output_mask,
           Wm0, Ws0, bs0, Wo0, Wm1, Ws1, bs1, Wo1):
    src = edge_index[0].astype(jnp.int32)
    dst = edge_index[1].astype(jnp.int32)
    mask = jnp.logical_and(set_mask, output_mask)
    zeros_acc = jnp.zeros((N, STATE), jnp.float32)
    state0, out0 = _gnn_layer(nodes, arcs, src, dst, edge_weights,
                              Wm0, Ws0, bs0, Wo0, mask, zeros_acc)
    nodes1 = jnp.concatenate([nodes, state0, out0], axis=1)
    _, out1 = _gnn_layer(nodes1, arcs, src, dst, edge_weights,
                         Wm1, Ws1, bs1, Wo1, mask, zeros_acc)
    return out1
